# baseline (speedup 1.0000x reference)
import functools

import jax
import jax.numpy as jnp
from jax import lax
from jax.experimental import pallas as pl
from jax.experimental.pallas import tpu as pltpu
from jax.experimental.pallas import tpu_sc as plsc

N = 50000
E = 800000
EL = 100000
D2 = 32
H = 64
OUT = 128

NC = 2
NS = 16
NPAD = 50176
TECROWS = NPAD // NS
NPROWS = NPAD // 4
EROWS = 6272
EPAD = EROWS * 128
RPC = 2
TEC_EROWS = EROWS // NS
CHUNKS = TEC_EROWS // RPC
ELPAD = 100352
ELTEC = ELPAD // (NC * NS)
NP8 = NPAD


def _mesh():
  return plsc.VectorSubcoreMesh(core_axis_name="c", subcore_axis_name="s",
                                num_cores=NC, num_subcores=NS)


def _perm(i):
  b = i // 1024
  w = i % 1024
  return b * 1024 + (w % 256) * 4 + w // 256


def _tc_embed(xt, xa, wt, bt, wa, ba):
  B = 1024

  def body(xt_ref, xa_ref, wt_ref, bt_ref, wa_ref, ba_ref, lo_ref, hi_ref):
    ylo = jnp.dot(xt_ref[...], wt_ref[...],
                  preferred_element_type=jnp.float32) + bt_ref[...]
    yhi = jnp.dot(xa_ref[...], wa_ref[...],
                  preferred_element_type=jnp.float32) + ba_ref[...]
    for a in range(4):
      lo_ref[:, 32 * a:32 * (a + 1)] = ylo[256 * a:256 * (a + 1), :]
      hi_ref[:, 32 * a:32 * (a + 1)] = yhi[256 * a:256 * (a + 1), :]

  return pl.pallas_call(
      body,
      grid=(NPAD // B,),
      in_specs=[
          pl.BlockSpec((B, 768), lambda i: (i, 0)),
          pl.BlockSpec((B, 768), lambda i: (i, 0)),
          pl.BlockSpec((768, D2), lambda i: (0, 0)),
          pl.BlockSpec((1, D2), lambda i: (0, 0)),
          pl.BlockSpec((768, D2), lambda i: (0, 0)),
          pl.BlockSpec((1, D2), lambda i: (0, 0)),
      ],
      out_specs=[pl.BlockSpec((256, 128), lambda i: (i, 0))] * 2,
      out_shape=[jax.ShapeDtypeStruct((NPROWS, 128), jnp.float32)] * 2,
  )(xt, xa, wt, bt.reshape(1, D2), wa, ba.reshape(1, D2))


def _tc_sage(agglo, agghi, cnt0, cnt1, slo, shi, w_mats, b_lo, b_hi, relu):
  B = 256

  def body(al, ah, c0, c1, xl, xh, m0, m1, m2, m3, m4, m5, m6, m7, bl, bh,
           ol, oh):
    inv = 1.0 / jnp.maximum(c0[...] + c1[...], 1.0)
    ml = al[...] * inv
    mh = ah[...] * inv
    dot = lambda x, w: jnp.dot(x, w[...], preferred_element_type=jnp.float32)
    hl = dot(ml, m0) + dot(mh, m1) + dot(xl[...], m2) + dot(xh[...], m3) + bl[...]
    hh = dot(ml, m4) + dot(mh, m5) + dot(xl[...], m6) + dot(xh[...], m7) + bh[...]
    if relu:
      hl = jnp.maximum(hl, 0.0)
      hh = jnp.maximum(hh, 0.0)
    ol[...] = hl
    oh[...] = hh

  row = lambda i: (i, 0)
  fixed = lambda i: (0, 0)
  wspec = pl.BlockSpec((128, 128), fixed)
  return pl.pallas_call(
      body,
      grid=(NPROWS // B,),
      in_specs=[pl.BlockSpec((B, 128), row)] * 6 + [wspec] * 8 +
               [pl.BlockSpec((1, 128), fixed)] * 2,
      out_specs=[pl.BlockSpec((B, 128), row)] * 2,
      out_shape=[jax.ShapeDtypeStruct((NPROWS, 128), jnp.float32)] * 2,
  )(agglo, agghi, cnt0, cnt1, slo, shi, *w_mats, b_lo, b_hi)


def _sc_counts(sd2d):
  @functools.partial(
      pl.kernel,
      out_type=(jax.ShapeDtypeStruct((NPAD, D2), jnp.float32),
                jax.ShapeDtypeStruct((NPAD, D2), jnp.float32)),
      mesh=_mesh(),
      compiler_params=pltpu.CompilerParams(use_tc_tiling_on_sc=False),
      scratch_types=[
          pltpu.VMEM_SHARED((NPAD, D2), jnp.float32),
          pltpu.VMEM((128, D2), jnp.float32),
          pltpu.VMEM((4, 2, 128), jnp.int32),
          pltpu.VMEM((196, D2), jnp.float32),
      ],
  )
  def run(dst_hbm, out0, out1, cnt, ones, didx, zbuf):
    c = lax.axis_index("c")
    s = lax.axis_index("s")
    one = jnp.ones((16,), jnp.float32)
    zero = jnp.zeros((16,), jnp.float32)

    def fill_ones(i, carry):
      ones[i, pl.ds(0, 16)] = one
      ones[i, pl.ds(16, 16)] = one
      return carry

    lax.fori_loop(0, 128, fill_ones, 0)

    def fill_zero(i, carry):
      zbuf[i, pl.ds(0, 16)] = zero
      zbuf[i, pl.ds(16, 16)] = zero
      return carry

    lax.fori_loop(0, 196, fill_zero, 0)

    def zero_acc(t, carry):
      pltpu.sync_copy(zbuf, cnt.at[pl.ds(s * TECROWS + t * 196, 196), :])
      return carry

    lax.fori_loop(0, 16, zero_acc, 0)
    plsc.subcore_barrier()

    base = (c * NS + s) * (EROWS // (NC * NS))

    def chunk(k, carry):
      pltpu.sync_copy(dst_hbm.at[pl.ds(base + k * 4, 4), :, :], didx)
      for j in range(4):
        pltpu.sync_copy(ones, cnt.at[didx.at[j, 1]], add=True)
      return carry

    lax.fori_loop(0, 49, chunk, 0)
    plsc.subcore_barrier()

    @pl.when(c == 0)
    def _():
      pltpu.sync_copy(cnt.at[pl.ds(s * TECROWS, TECROWS), :],
                      out0.at[pl.ds(s * TECROWS, TECROWS), :])

    @pl.when(c == 1)
    def _():
      pltpu.sync_copy(cnt.at[pl.ds(s * TECROWS, TECROWS), :],
                      out1.at[pl.ds(s * TECROWS, TECROWS), :])

  return run(sd2d)


def _sc_scatter(sd2d, tlo, thi):
  @functools.partial(
      pl.kernel,
      out_type=(jax.ShapeDtypeStruct((NPAD, D2), jnp.float32),
                jax.ShapeDtypeStruct((NPAD, D2), jnp.float32)),
      mesh=_mesh(),
      compiler_params=pltpu.CompilerParams(use_tc_tiling_on_sc=False),
      scratch_types=[
          pltpu.VMEM_SHARED((NPAD, D2), jnp.float32),
          pltpu.VMEM((4, RPC, 2, 128), jnp.int32),
          pltpu.VMEM((2, RPC * 128, D2), jnp.float32),
          pltpu.SemaphoreType.DMA,
          pltpu.SemaphoreType.DMA,
          pltpu.SemaphoreType.DMA,
      ],
  )
  def run(sd_hbm, tlo_hbm, thi_hbm, out_lo, out_hi,
          acc, idx, rows, isem, gsem, ssem):
    c = lax.axis_index("c")
    s = lax.axis_index("s")
    zero = jnp.zeros((16,), jnp.float32)

    def fill_zero(i, carry):
      rows[0, i, pl.ds(0, 16)] = zero
      rows[0, i, pl.ds(16, 16)] = zero
      rows[1, i, pl.ds(0, 16)] = zero
      rows[1, i, pl.ds(16, 16)] = zero
      return carry

    lax.fori_loop(0, RPC * 128, fill_zero, 0)
    nz = TECROWS // (RPC * 128)

    def zero_acc(t, carry):
      pltpu.sync_copy(rows.at[0],
                      acc.at[pl.ds(s * TECROWS + t * (RPC * 128), RPC * 128), :])
      return carry

    lax.fori_loop(0, nz, zero_acc, 0)
    rem = TECROWS - nz * RPC * 128
    if rem:
      pltpu.sync_copy(rows.at[1, pl.ds(0, rem), :],
                      acc.at[pl.ds(s * TECROWS + nz * RPC * 128, rem), :])
    plsc.subcore_barrier()

    base = s * TEC_EROWS

    def fire_idx(k):
      pltpu.async_copy(sd_hbm.at[pl.ds(base + k * RPC, RPC), :, :],
                       idx.at[lax.rem(k, 4)], isem)

    def wait_idx(k):
      pltpu.make_async_copy(sd_hbm.at[pl.ds(0, RPC), :, :],
                            idx.at[lax.rem(k, 4)], isem).wait()

    def fire_gathers(tbl, k):
      s4 = lax.rem(k, 4)
      s2 = lax.rem(k, 2)
      for j in range(RPC):
        pltpu.async_copy(tbl.at[idx.at[s4, j, 0]],
                         rows.at[s2, pl.ds(j * 128, 128), :], gsem)

    def fire_gathers_c(k):
      @pl.when(c == 0)
      def _():
        fire_gathers(tlo_hbm, k)

      @pl.when(c == 1)
      def _():
        fire_gathers(thi_hbm, k)

    def drain(k, sem):
      s2 = lax.rem(k, 2)
      pltpu.make_async_copy(tlo_hbm.at[pl.ds(0, RPC * 128), :],
                            rows.at[s2], sem).wait()

    def fire_scatters(k):
      s4 = lax.rem(k, 4)
      s2 = lax.rem(k, 2)
      for j in range(RPC):
        pltpu.async_copy(rows.at[s2, pl.ds(j * 128, 128), :],
                         acc.at[idx.at[s4, j, 1]], ssem, add=True)

    fire_idx(0)
    fire_idx(1)
    wait_idx(0)
    fire_gathers_c(0)

    def body(k, carry):
      drain(k, gsem)

      @pl.when(k >= 1)
      def _():
        drain(k - 1, ssem)

      @pl.when(k + 2 < CHUNKS)
      def _():
        fire_idx(k + 2)

      @pl.when(k + 1 < CHUNKS)
      def _():
        wait_idx(k + 1)
        fire_gathers_c(k + 1)

      fire_scatters(k)
      return carry

    lax.fori_loop(0, CHUNKS, body, 0)
    drain(CHUNKS - 1, ssem)
    plsc.subcore_barrier()

    @pl.when(c == 0)
    def _():
      pltpu.sync_copy(acc.at[pl.ds(s * TECROWS, TECROWS), :],
                      out_lo.at[pl.ds(s * TECROWS, TECROWS), :])

    @pl.when(c == 1)
    def _():
      pltpu.sync_copy(acc.at[pl.ds(s * TECROWS, TECROWS), :],
                      out_hi.at[pl.ds(s * TECROWS, TECROWS), :])

  return run(sd2d, tlo, thi)


def _sc_edge_gather(p_pad, q_pad, i0, i1):
  @functools.partial(
      pl.kernel,
      out_type=jax.ShapeDtypeStruct((ELPAD,), jnp.float32),
      mesh=_mesh(),
      compiler_params=pltpu.CompilerParams(use_tc_tiling_on_sc=False,
                                           needs_layout_passes=False),
      scratch_types=[
          pltpu.VMEM((NP8,), jnp.float32),
          pltpu.VMEM((NP8,), jnp.float32),
          pltpu.VMEM((ELTEC,), jnp.int32),
          pltpu.VMEM((ELTEC,), jnp.int32),
          pltpu.VMEM((ELTEC,), jnp.float32),
      ],
  )
  def run(p_hbm, q_hbm, i0_hbm, i1_hbm, out_hbm, pv, qv, i0b, i1b, ob):
    c = lax.axis_index("c")
    s = lax.axis_index("s")
    base = (s * NC + c) * ELTEC
    pltpu.sync_copy(p_hbm, pv)
    pltpu.sync_copy(q_hbm, qv)
    pltpu.sync_copy(i0_hbm.at[pl.ds(base, ELTEC)], i0b)
    pltpu.sync_copy(i1_hbm.at[pl.ds(base, ELTEC)], i1b)

    def step(j, carry):
      ia = i0b[pl.ds(j * 16, 16)]
      ib = i1b[pl.ds(j * 16, 16)]
      ob[pl.ds(j * 16, 16)] = (plsc.load_gather(pv, [ia])
                               + plsc.load_gather(qv, [ib]))
      return carry

    lax.fori_loop(0, ELTEC // 16, step, 0)
    pltpu.sync_copy(ob, out_hbm.at[pl.ds(base, ELTEC)])

  return run(p_pad, q_pad, i0, i1)


def _bd(w32):
  return jnp.kron(jnp.eye(4, dtype=jnp.float32), w32)


def kernel(x_titles_inputs, x_abstracts_inputs, edge_index, edge_label_index,
           Wt, bt, Wa, ba, W1l, W1r, b1, W2l, W2r, b2, Wc, bc):
  src = _perm(edge_index[0])
  dst = _perm(edge_index[1])
  padn = EPAD - E
  pad_src = (jnp.arange(padn, dtype=jnp.int32) * 17) % N
  pad_dst = _perm(N + jnp.arange(padn, dtype=jnp.int32) % (NPAD - N))
  src2d = jnp.concatenate([src, pad_src]).reshape(EROWS, 128)
  dst2d = jnp.concatenate([dst, pad_dst]).reshape(EROWS, 128)
  sd2d = jnp.stack([src2d, dst2d], axis=1)

  xlo, xhi = _tc_embed(x_titles_inputs, x_abstracts_inputs, Wt, bt, Wa, ba)

  cnt0, cnt1 = _sc_counts(sd2d)
  cp0 = cnt0.reshape(NPROWS, 128)
  cp1 = cnt1.reshape(NPROWS, 128)

  agg1lo, agg1hi = _sc_scatter(sd2d, xlo.reshape(NPAD, D2),
                               xhi.reshape(NPAD, D2))

  w1 = [_bd(W1l[:D2, :D2]), _bd(W1l[D2:, :D2]),
        _bd(W1r[:D2, :D2]), _bd(W1r[D2:, :D2]),
        _bd(W1l[:D2, D2:]), _bd(W1l[D2:, D2:]),
        _bd(W1r[:D2, D2:]), _bd(W1r[D2:, D2:])]
  b1lo = jnp.tile(b1[:D2], 4).reshape(1, 128)
  b1hi = jnp.tile(b1[D2:], 4).reshape(1, 128)
  h1lo, h1hi = _tc_sage(agg1lo.reshape(NPROWS, 128),
                        agg1hi.reshape(NPROWS, 128),
                        cp0, cp1, xlo, xhi, w1, b1lo, b1hi, relu=True)

  agg2lo, agg2hi = _sc_scatter(sd2d, h1lo.reshape(NPAD, D2),
                               h1hi.reshape(NPAD, D2))

  wcu = Wc[:OUT]
  wcv = Wc[OUT:]
  u = W2l @ wcu
  v = W2l @ wcv
  r_ = W2r @ wcu
  t_ = W2r @ wcv

  def arrange(col_u, col_v):
    base = jnp.zeros((D2, D2), jnp.float32)
    base = base.at[:, 0].set(col_u[:, 0]).at[:, 16].set(col_v[:, 0])
    return _bd(base)

  zmat = jnp.zeros((128, 128), jnp.float32)
  w2m = [arrange(u[:D2], v[:D2]), arrange(u[D2:], v[D2:]),
         arrange(r_[:D2], t_[:D2]), arrange(r_[D2:], t_[D2:]),
         zmat, zmat, zmat, zmat]
  c_p = (b2 @ wcu + bc)[0]
  c_q = (b2 @ wcv)[0]
  cvec = jnp.zeros((D2,), jnp.float32).at[0].set(c_p).at[16].set(c_q)
  cvec = jnp.tile(cvec, 4).reshape(1, 128)
  pq, _ = _tc_sage(agg2lo.reshape(NPROWS, 128),
                   agg2hi.reshape(NPROWS, 128),
                   cp0, cp1, h1lo, h1hi, w2m, cvec,
                   jnp.zeros((1, 128), jnp.float32), relu=False)

  o = pq.reshape(NPAD, D2)
  p_pad = o[:, 0]
  q_pad = o[:, 16]
  i0 = jnp.pad(_perm(edge_label_index[0]), (0, ELPAD - EL))
  i1 = jnp.pad(_perm(edge_label_index[1]), (0, ELPAD - EL))
  outf = _sc_edge_gather(p_pad, q_pad, i0, i1)
  return outf[:EL].reshape(EL, 1)


# speedup vs baseline: 1.0879x; 1.0879x over previous
import functools

import jax
import jax.numpy as jnp
from jax import lax
from jax.experimental import pallas as pl
from jax.experimental.pallas import tpu as pltpu
from jax.experimental.pallas import tpu_sc as plsc

N = 50000
E = 800000
EL = 100000
D2 = 32
H = 64
OUT = 128

NC = 2
NS = 16
NPAD = 50176
TECROWS = NPAD // NS
NPROWS = NPAD // 4
EROWS = 6272
EPAD = EROWS * 128
RPC = 2
TEC_EROWS = EROWS // NS
CHUNKS = TEC_EROWS // RPC
ELPAD = 100352
ELTEC = ELPAD // (NC * NS)
NP8 = NPAD


def _mesh():
  return plsc.VectorSubcoreMesh(core_axis_name="c", subcore_axis_name="s",
                                num_cores=NC, num_subcores=NS)


def _perm(i):
  b = i // 1024
  w = i % 1024
  return b * 1024 + (w % 256) * 4 + w // 256


def _tc_embed(xt, xa, wt, bt, wa, ba):
  B = 1024

  def body(xt_ref, xa_ref, wt_ref, bt_ref, wa_ref, ba_ref, lo_ref, hi_ref):
    ylo = jnp.dot(xt_ref[...], wt_ref[...],
                  preferred_element_type=jnp.float32) + bt_ref[...]
    yhi = jnp.dot(xa_ref[...], wa_ref[...],
                  preferred_element_type=jnp.float32) + ba_ref[...]
    for a in range(4):
      lo_ref[:, 32 * a:32 * (a + 1)] = ylo[256 * a:256 * (a + 1), :]
      hi_ref[:, 32 * a:32 * (a + 1)] = yhi[256 * a:256 * (a + 1), :]

  return pl.pallas_call(
      body,
      grid=(NPAD // B,),
      in_specs=[
          pl.BlockSpec((B, 768), lambda i: (i, 0)),
          pl.BlockSpec((B, 768), lambda i: (i, 0)),
          pl.BlockSpec((768, D2), lambda i: (0, 0)),
          pl.BlockSpec((1, D2), lambda i: (0, 0)),
          pl.BlockSpec((768, D2), lambda i: (0, 0)),
          pl.BlockSpec((1, D2), lambda i: (0, 0)),
      ],
      out_specs=[pl.BlockSpec((256, 128), lambda i: (i, 0))] * 2,
      out_shape=[jax.ShapeDtypeStruct((NPROWS, 128), jnp.float32)] * 2,
  )(xt, xa, wt, bt.reshape(1, D2), wa, ba.reshape(1, D2))


def _tc_sage(agglo, agghi, cntp, slo, shi, w_mats, b_lo, b_hi, relu):
  B = 256

  def body(al, ah, ct, xl, xh, m0, m1, m2, m3, m4, m5, m6, m7, bl, bh,
           ol, oh):
    inv = 1.0 / jnp.maximum(ct[...], 1.0)
    ml = al[...] * inv
    mh = ah[...] * inv
    dot = lambda x, w: jnp.dot(x, w[...], preferred_element_type=jnp.float32)
    hl = dot(ml, m0) + dot(mh, m1) + dot(xl[...], m2) + dot(xh[...], m3) + bl[...]
    hh = dot(ml, m4) + dot(mh, m5) + dot(xl[...], m6) + dot(xh[...], m7) + bh[...]
    if relu:
      hl = jnp.maximum(hl, 0.0)
      hh = jnp.maximum(hh, 0.0)
    ol[...] = hl
    oh[...] = hh

  row = lambda i: (i, 0)
  fixed = lambda i: (0, 0)
  wspec = pl.BlockSpec((128, 128), fixed)
  return pl.pallas_call(
      body,
      grid=(NPROWS // B,),
      in_specs=[pl.BlockSpec((B, 128), row)] * 5 + [wspec] * 8 +
               [pl.BlockSpec((1, 128), fixed)] * 2,
      out_specs=[pl.BlockSpec((B, 128), row)] * 2,
      out_shape=[jax.ShapeDtypeStruct((NPROWS, 128), jnp.float32)] * 2,
  )(agglo, agghi, cntp, slo, shi, *w_mats, b_lo, b_hi)


def _sc_counts(sd2d):
  @functools.partial(
      pl.kernel,
      out_type=(jax.ShapeDtypeStruct((NPAD, 16), jnp.float32),
                jax.ShapeDtypeStruct((NPAD, 16), jnp.float32)),
      mesh=_mesh(),
      compiler_params=pltpu.CompilerParams(use_tc_tiling_on_sc=False),
      scratch_types=[
          pltpu.VMEM_SHARED((NPAD, 16), jnp.float32),
          pltpu.VMEM((128, 16), jnp.float32),
          pltpu.VMEM((4, 2, 128), jnp.int32),
          pltpu.VMEM((196, 16), jnp.float32),
      ],
  )
  def run(dst_hbm, out0, out1, cnt, ones, didx, zbuf):
    c = lax.axis_index("c")
    s = lax.axis_index("s")
    one = jnp.ones((16,), jnp.float32)
    zero = jnp.zeros((16,), jnp.float32)

    def fill_ones(i, carry):
      ones[i, pl.ds(0, 16)] = one
      return carry

    lax.fori_loop(0, 128, fill_ones, 0)

    def fill_zero(i, carry):
      zbuf[i, pl.ds(0, 16)] = zero
      return carry

    lax.fori_loop(0, 196, fill_zero, 0)

    def zero_acc(t, carry):
      pltpu.sync_copy(zbuf, cnt.at[pl.ds(s * TECROWS + t * 196, 196), :])
      return carry

    lax.fori_loop(0, 16, zero_acc, 0)
    plsc.subcore_barrier()

    base = (c * NS + s) * (EROWS // (NC * NS))

    def chunk(k, carry):
      pltpu.sync_copy(dst_hbm.at[pl.ds(base + k * 4, 4), :, :], didx)
      for j in range(4):
        pltpu.sync_copy(ones, cnt.at[didx.at[j, 1]], add=True)
      return carry

    lax.fori_loop(0, 49, chunk, 0)
    plsc.subcore_barrier()

    @pl.when(c == 0)
    def _():
      pltpu.sync_copy(cnt.at[pl.ds(s * TECROWS, TECROWS), :],
                      out0.at[pl.ds(s * TECROWS, TECROWS), :])

    @pl.when(c == 1)
    def _():
      pltpu.sync_copy(cnt.at[pl.ds(s * TECROWS, TECROWS), :],
                      out1.at[pl.ds(s * TECROWS, TECROWS), :])

  return run(sd2d)


def _sc_scatter(sd2d, tlo, thi):
  @functools.partial(
      pl.kernel,
      out_type=(jax.ShapeDtypeStruct((NPAD, D2), jnp.float32),
                jax.ShapeDtypeStruct((NPAD, D2), jnp.float32)),
      mesh=_mesh(),
      compiler_params=pltpu.CompilerParams(use_tc_tiling_on_sc=False),
      scratch_types=[
          pltpu.VMEM_SHARED((NPAD, D2), jnp.float32),
          pltpu.VMEM((4, RPC, 2, 128), jnp.int32),
          pltpu.VMEM((2, RPC * 128, D2), jnp.float32),
          pltpu.SemaphoreType.DMA,
          pltpu.SemaphoreType.DMA,
          pltpu.SemaphoreType.DMA,
      ],
  )
  def run(sd_hbm, tlo_hbm, thi_hbm, out_lo, out_hi,
          acc, idx, rows, isem, gsem, ssem):
    c = lax.axis_index("c")
    s = lax.axis_index("s")
    zero = jnp.zeros((16,), jnp.float32)

    def fill_zero(i, carry):
      rows[0, i, pl.ds(0, 16)] = zero
      rows[0, i, pl.ds(16, 16)] = zero
      rows[1, i, pl.ds(0, 16)] = zero
      rows[1, i, pl.ds(16, 16)] = zero
      return carry

    lax.fori_loop(0, RPC * 128, fill_zero, 0)
    nz = TECROWS // (RPC * 128)

    def zero_acc(t, carry):
      pltpu.sync_copy(rows.at[0],
                      acc.at[pl.ds(s * TECROWS + t * (RPC * 128), RPC * 128), :])
      return carry

    lax.fori_loop(0, nz, zero_acc, 0)
    rem = TECROWS - nz * RPC * 128
    if rem:
      pltpu.sync_copy(rows.at[1, pl.ds(0, rem), :],
                      acc.at[pl.ds(s * TECROWS + nz * RPC * 128, rem), :])
    plsc.subcore_barrier()

    base = s * TEC_EROWS

    def fire_idx(k):
      pltpu.async_copy(sd_hbm.at[pl.ds(base + k * RPC, RPC), :, :],
                       idx.at[lax.rem(k, 4)], isem)

    def wait_idx(k):
      pltpu.make_async_copy(sd_hbm.at[pl.ds(0, RPC), :, :],
                            idx.at[lax.rem(k, 4)], isem).wait()

    def fire_gathers(tbl, k):
      s4 = lax.rem(k, 4)
      s2 = lax.rem(k, 2)
      for j in range(RPC):
        pltpu.async_copy(tbl.at[idx.at[s4, j, 0]],
                         rows.at[s2, pl.ds(j * 128, 128), :], gsem)

    def fire_gathers_c(k):
      @pl.when(c == 0)
      def _():
        fire_gathers(tlo_hbm, k)

      @pl.when(c == 1)
      def _():
        fire_gathers(thi_hbm, k)

    def drain(k, sem):
      s2 = lax.rem(k, 2)
      pltpu.make_async_copy(tlo_hbm.at[pl.ds(0, RPC * 128), :],
                            rows.at[s2], sem).wait()

    def fire_scatters(k):
      s4 = lax.rem(k, 4)
      s2 = lax.rem(k, 2)
      for j in range(RPC):
        pltpu.async_copy(rows.at[s2, pl.ds(j * 128, 128), :],
                         acc.at[idx.at[s4, j, 1]], ssem, add=True)

    fire_idx(0)
    fire_idx(1)
    wait_idx(0)
    fire_gathers_c(0)

    def body(k, carry):
      drain(k, gsem)

      @pl.when(k >= 1)
      def _():
        drain(k - 1, ssem)

      @pl.when(k + 2 < CHUNKS)
      def _():
        fire_idx(k + 2)

      @pl.when(k + 1 < CHUNKS)
      def _():
        wait_idx(k + 1)
        fire_gathers_c(k + 1)

      fire_scatters(k)
      return carry

    lax.fori_loop(0, CHUNKS, body, 0)
    drain(CHUNKS - 1, ssem)
    plsc.subcore_barrier()

    @pl.when(c == 0)
    def _():
      pltpu.sync_copy(acc.at[pl.ds(s * TECROWS, TECROWS), :],
                      out_lo.at[pl.ds(s * TECROWS, TECROWS), :])

    @pl.when(c == 1)
    def _():
      pltpu.sync_copy(acc.at[pl.ds(s * TECROWS, TECROWS), :],
                      out_hi.at[pl.ds(s * TECROWS, TECROWS), :])

  return run(sd2d, tlo, thi)


def _sc_edge_gather(p_pad, q_pad, i0, i1):
  @functools.partial(
      pl.kernel,
      out_type=jax.ShapeDtypeStruct((ELPAD,), jnp.float32),
      mesh=_mesh(),
      compiler_params=pltpu.CompilerParams(use_tc_tiling_on_sc=False,
                                           needs_layout_passes=False),
      scratch_types=[
          pltpu.VMEM((NP8,), jnp.float32),
          pltpu.VMEM((NP8,), jnp.float32),
          pltpu.VMEM((ELTEC,), jnp.int32),
          pltpu.VMEM((ELTEC,), jnp.int32),
          pltpu.VMEM((ELTEC,), jnp.float32),
      ],
  )
  def run(p_hbm, q_hbm, i0_hbm, i1_hbm, out_hbm, pv, qv, i0b, i1b, ob):
    c = lax.axis_index("c")
    s = lax.axis_index("s")
    base = (s * NC + c) * ELTEC
    pltpu.sync_copy(p_hbm, pv)
    pltpu.sync_copy(q_hbm, qv)
    pltpu.sync_copy(i0_hbm.at[pl.ds(base, ELTEC)], i0b)
    pltpu.sync_copy(i1_hbm.at[pl.ds(base, ELTEC)], i1b)

    def step(j, carry):
      ia = i0b[pl.ds(j * 16, 16)]
      ib = i1b[pl.ds(j * 16, 16)]
      ob[pl.ds(j * 16, 16)] = (plsc.load_gather(pv, [ia])
                               + plsc.load_gather(qv, [ib]))
      return carry

    lax.fori_loop(0, ELTEC // 16, step, 0)
    pltpu.sync_copy(ob, out_hbm.at[pl.ds(base, ELTEC)])

  return run(p_pad, q_pad, i0, i1)


def _bd(w32):
  return jnp.kron(jnp.eye(4, dtype=jnp.float32), w32)


def kernel(x_titles_inputs, x_abstracts_inputs, edge_index, edge_label_index,
           Wt, bt, Wa, ba, W1l, W1r, b1, W2l, W2r, b2, Wc, bc):
  src = _perm(edge_index[0])
  dst = _perm(edge_index[1])
  padn = EPAD - E
  pad_src = (jnp.arange(padn, dtype=jnp.int32) * 17) % N
  pad_dst = _perm(N + jnp.arange(padn, dtype=jnp.int32) % (NPAD - N))
  src2d = jnp.concatenate([src, pad_src]).reshape(EROWS, 128)
  dst2d = jnp.concatenate([dst, pad_dst]).reshape(EROWS, 128)
  sd2d = jnp.stack([src2d, dst2d], axis=1)

  xlo, xhi = _tc_embed(x_titles_inputs, x_abstracts_inputs, Wt, bt, Wa, ba)

  cnt0, cnt1 = _sc_counts(sd2d)
  cnt1d = cnt0[:, 0] + cnt1[:, 0]
  cntp = jnp.repeat(cnt1d.reshape(NPROWS, 4), D2, axis=1)

  agg1lo, agg1hi = _sc_scatter(sd2d, xlo.reshape(NPAD, D2),
                               xhi.reshape(NPAD, D2))

  w1 = [_bd(W1l[:D2, :D2]), _bd(W1l[D2:, :D2]),
        _bd(W1r[:D2, :D2]), _bd(W1r[D2:, :D2]),
        _bd(W1l[:D2, D2:]), _bd(W1l[D2:, D2:]),
        _bd(W1r[:D2, D2:]), _bd(W1r[D2:, D2:])]
  b1lo = jnp.tile(b1[:D2], 4).reshape(1, 128)
  b1hi = jnp.tile(b1[D2:], 4).reshape(1, 128)
  h1lo, h1hi = _tc_sage(agg1lo.reshape(NPROWS, 128),
                        agg1hi.reshape(NPROWS, 128),
                        cntp, xlo, xhi, w1, b1lo, b1hi, relu=True)

  agg2lo, agg2hi = _sc_scatter(sd2d, h1lo.reshape(NPAD, D2),
                               h1hi.reshape(NPAD, D2))

  wcu = Wc[:OUT]
  wcv = Wc[OUT:]
  u = W2l @ wcu
  v = W2l @ wcv
  r_ = W2r @ wcu
  t_ = W2r @ wcv

  def arrange(col_u, col_v):
    base = jnp.zeros((D2, D2), jnp.float32)
    base = base.at[:, 0].set(col_u[:, 0]).at[:, 16].set(col_v[:, 0])
    return _bd(base)

  zmat = jnp.zeros((128, 128), jnp.float32)
  w2m = [arrange(u[:D2], v[:D2]), arrange(u[D2:], v[D2:]),
         arrange(r_[:D2], t_[:D2]), arrange(r_[D2:], t_[D2:]),
         zmat, zmat, zmat, zmat]
  c_p = (b2 @ wcu + bc)[0]
  c_q = (b2 @ wcv)[0]
  cvec = jnp.zeros((D2,), jnp.float32).at[0].set(c_p).at[16].set(c_q)
  cvec = jnp.tile(cvec, 4).reshape(1, 128)
  pq, _ = _tc_sage(agg2lo.reshape(NPROWS, 128),
                   agg2hi.reshape(NPROWS, 128),
                   cntp, h1lo, h1hi, w2m, cvec,
                   jnp.zeros((1, 128), jnp.float32), relu=False)

  o = pq.reshape(NPAD, D2)
  p_pad = o[:, 0]
  q_pad = o[:, 16]
  i0 = jnp.pad(_perm(edge_label_index[0]), (0, ELPAD - EL))
  i1 = jnp.pad(_perm(edge_label_index[1]), (0, ELPAD - EL))
  outf = _sc_edge_gather(p_pad, q_pad, i0, i1)
  return outf[:EL].reshape(EL, 1)


# speedup vs baseline: 1.1202x; 1.0296x over previous
import functools

import jax
import jax.numpy as jnp
from jax import lax
from jax.experimental import pallas as pl
from jax.experimental.pallas import tpu as pltpu
from jax.experimental.pallas import tpu_sc as plsc

N = 50000
E = 800000
EL = 100000
D2 = 32
H = 64
OUT = 128

NC = 2
NS = 16
NPAD = 50176
TECROWS = NPAD // NS
NPROWS = NPAD // 4
EROWS = 6272
EPAD = EROWS * 128
RPC = 2
TEC_EROWS = EROWS // NS
CHUNKS = TEC_EROWS // RPC
ELPAD = 100352
ELTEC = ELPAD // (NC * NS)
NP8 = NPAD


def _mesh():
  return plsc.VectorSubcoreMesh(core_axis_name="c", subcore_axis_name="s",
                                num_cores=NC, num_subcores=NS)


def _perm(i):
  b = i // 1024
  w = i % 1024
  return b * 1024 + (w % 256) * 4 + w // 256


def _tc_embed(xt, xa, wt, bt, wa, ba):
  B = 1024

  def body(xt_ref, xa_ref, wt_ref, bt_ref, wa_ref, ba_ref, lo_ref, hi_ref):
    ylo = jnp.dot(xt_ref[...], wt_ref[...],
                  preferred_element_type=jnp.float32) + bt_ref[...]
    yhi = jnp.dot(xa_ref[...], wa_ref[...],
                  preferred_element_type=jnp.float32) + ba_ref[...]
    for a in range(4):
      lo_ref[:, 32 * a:32 * (a + 1)] = ylo[256 * a:256 * (a + 1), :]
      hi_ref[:, 32 * a:32 * (a + 1)] = yhi[256 * a:256 * (a + 1), :]

  return pl.pallas_call(
      body,
      grid=(NPAD // B,),
      in_specs=[
          pl.BlockSpec((B, 768), lambda i: (i, 0)),
          pl.BlockSpec((B, 768), lambda i: (i, 0)),
          pl.BlockSpec((768, D2), lambda i: (0, 0)),
          pl.BlockSpec((1, D2), lambda i: (0, 0)),
          pl.BlockSpec((768, D2), lambda i: (0, 0)),
          pl.BlockSpec((1, D2), lambda i: (0, 0)),
      ],
      out_specs=[pl.BlockSpec((256, 128), lambda i: (i, 0))] * 2,
      out_shape=[jax.ShapeDtypeStruct((NPROWS, 128), jnp.float32)] * 2,
  )(xt, xa, wt, bt.reshape(1, D2), wa, ba.reshape(1, D2))


def _tc_sage(agglo, agghi, cntp, slo, shi, w_mats, b_lo, b_hi, relu):
  B = 256

  def body(al, ah, ct, xl, xh, m0, m1, m2, m3, m4, m5, m6, m7, bl, bh,
           ol, oh):
    inv = 1.0 / jnp.maximum(ct[...], 1.0)
    ml = al[...] * inv
    mh = ah[...] * inv
    dot = lambda x, w: jnp.dot(x, w[...], preferred_element_type=jnp.float32)
    hl = dot(ml, m0) + dot(mh, m1) + dot(xl[...], m2) + dot(xh[...], m3) + bl[...]
    hh = dot(ml, m4) + dot(mh, m5) + dot(xl[...], m6) + dot(xh[...], m7) + bh[...]
    if relu:
      hl = jnp.maximum(hl, 0.0)
      hh = jnp.maximum(hh, 0.0)
    ol[...] = hl
    oh[...] = hh

  row = lambda i: (i, 0)
  fixed = lambda i: (0, 0)
  wspec = pl.BlockSpec((128, 128), fixed)
  return pl.pallas_call(
      body,
      grid=(NPROWS // B,),
      in_specs=[pl.BlockSpec((B, 128), row)] * 5 + [wspec] * 8 +
               [pl.BlockSpec((1, 128), fixed)] * 2,
      out_specs=[pl.BlockSpec((B, 128), row)] * 2,
      out_shape=[jax.ShapeDtypeStruct((NPROWS, 128), jnp.float32)] * 2,
  )(agglo, agghi, cntp, slo, shi, *w_mats, b_lo, b_hi)


def _sc_counts(sd2d):
  @functools.partial(
      pl.kernel,
      out_type=(jax.ShapeDtypeStruct((NPAD, 16), jnp.float32),
                jax.ShapeDtypeStruct((NPAD, 16), jnp.float32)),
      mesh=_mesh(),
      compiler_params=pltpu.CompilerParams(use_tc_tiling_on_sc=False),
      scratch_types=[
          pltpu.VMEM_SHARED((NPAD, 16), jnp.float32),
          pltpu.VMEM((128, 16), jnp.float32),
          pltpu.VMEM((4, 2, 128), jnp.int32),
          pltpu.VMEM((196, 16), jnp.float32),
      ],
  )
  def run(dst_hbm, out0, out1, cnt, ones, didx, zbuf):
    c = lax.axis_index("c")
    s = lax.axis_index("s")
    one = jnp.ones((16,), jnp.float32)
    zero = jnp.zeros((16,), jnp.float32)

    def fill_ones(i, carry):
      ones[i, pl.ds(0, 16)] = one
      return carry

    lax.fori_loop(0, 128, fill_ones, 0)

    def fill_zero(i, carry):
      zbuf[i, pl.ds(0, 16)] = zero
      return carry

    lax.fori_loop(0, 196, fill_zero, 0)

    def zero_acc(t, carry):
      pltpu.sync_copy(zbuf, cnt.at[pl.ds(s * TECROWS + t * 196, 196), :])
      return carry

    lax.fori_loop(0, 16, zero_acc, 0)
    plsc.subcore_barrier()

    base = (c * NS + s) * (EROWS // (NC * NS))

    def chunk(k, carry):
      pltpu.sync_copy(dst_hbm.at[pl.ds(base + k * 4, 4), :, :], didx)
      for j in range(4):
        pltpu.sync_copy(ones, cnt.at[didx.at[j, 1]], add=True)
      return carry

    lax.fori_loop(0, 49, chunk, 0)
    plsc.subcore_barrier()

    @pl.when(c == 0)
    def _():
      pltpu.sync_copy(cnt.at[pl.ds(s * TECROWS, TECROWS), :],
                      out0.at[pl.ds(s * TECROWS, TECROWS), :])

    @pl.when(c == 1)
    def _():
      pltpu.sync_copy(cnt.at[pl.ds(s * TECROWS, TECROWS), :],
                      out1.at[pl.ds(s * TECROWS, TECROWS), :])

  return run(sd2d)


def _sc_scatter(sd2d, tlo, thi):
  @functools.partial(
      pl.kernel,
      out_type=(jax.ShapeDtypeStruct((NPAD, D2), jnp.float32),
                jax.ShapeDtypeStruct((NPAD, D2), jnp.float32)),
      mesh=_mesh(),
      compiler_params=pltpu.CompilerParams(use_tc_tiling_on_sc=False),
      scratch_types=[
          pltpu.VMEM_SHARED((NPAD, D2), jnp.float32),
          pltpu.VMEM((4, RPC, 2, 128), jnp.int32),
          pltpu.VMEM((2, RPC * 128, D2), jnp.float32),
          pltpu.SemaphoreType.DMA,
          pltpu.SemaphoreType.DMA,
          pltpu.SemaphoreType.DMA,
      ],
  )
  def run(sd_hbm, tlo_hbm, thi_hbm, out_lo, out_hi,
          acc, idx, rows, isem, gsem, ssem):
    c = lax.axis_index("c")
    s = lax.axis_index("s")
    zero = jnp.zeros((16,), jnp.float32)

    def fill_zero(i, carry):
      rows[0, i, pl.ds(0, 16)] = zero
      rows[0, i, pl.ds(16, 16)] = zero
      rows[1, i, pl.ds(0, 16)] = zero
      rows[1, i, pl.ds(16, 16)] = zero
      return carry

    lax.fori_loop(0, RPC * 128, fill_zero, 0)
    nz = TECROWS // (RPC * 128)

    def zero_acc(t, carry):
      pltpu.sync_copy(rows.at[0],
                      acc.at[pl.ds(s * TECROWS + t * (RPC * 128), RPC * 128), :])
      return carry

    lax.fori_loop(0, nz, zero_acc, 0)
    rem = TECROWS - nz * RPC * 128
    if rem:
      pltpu.sync_copy(rows.at[1, pl.ds(0, rem), :],
                      acc.at[pl.ds(s * TECROWS + nz * RPC * 128, rem), :])
    plsc.subcore_barrier()

    base = s * TEC_EROWS

    def fire_idx(k):
      pltpu.async_copy(sd_hbm.at[pl.ds(base + k * RPC, RPC), :, :],
                       idx.at[lax.rem(k, 4)], isem)

    def wait_idx(k):
      pltpu.make_async_copy(sd_hbm.at[pl.ds(0, RPC), :, :],
                            idx.at[lax.rem(k, 4)], isem).wait()

    def fire_gathers(tbl, k):
      s4 = lax.rem(k, 4)
      s2 = lax.rem(k, 2)
      for j in range(RPC):
        pltpu.async_copy(tbl.at[idx.at[s4, j, 0]],
                         rows.at[s2, pl.ds(j * 128, 128), :], gsem)

    def fire_gathers_c(k):
      @pl.when(c == 0)
      def _():
        fire_gathers(tlo_hbm, k)

      @pl.when(c == 1)
      def _():
        fire_gathers(thi_hbm, k)

    def drain(k, sem):
      s2 = lax.rem(k, 2)
      pltpu.make_async_copy(tlo_hbm.at[pl.ds(0, RPC * 128), :],
                            rows.at[s2], sem).wait()

    def fire_scatters(k):
      s4 = lax.rem(k, 4)
      s2 = lax.rem(k, 2)
      for j in range(RPC):
        pltpu.async_copy(rows.at[s2, pl.ds(j * 128, 128), :],
                         acc.at[idx.at[s4, j, 1]], ssem, add=True)

    fire_idx(0)
    fire_idx(1)
    wait_idx(0)
    fire_gathers_c(0)

    def body(k, carry):
      drain(k, gsem)

      @pl.when(k >= 1)
      def _():
        drain(k - 1, ssem)

      @pl.when(k + 2 < CHUNKS)
      def _():
        fire_idx(k + 2)

      @pl.when(k + 1 < CHUNKS)
      def _():
        wait_idx(k + 1)
        fire_gathers_c(k + 1)

      fire_scatters(k)
      return carry

    lax.fori_loop(0, CHUNKS, body, 0)
    drain(CHUNKS - 1, ssem)
    plsc.subcore_barrier()

    @pl.when(c == 0)
    def _():
      pltpu.sync_copy(acc.at[pl.ds(s * TECROWS, TECROWS), :],
                      out_lo.at[pl.ds(s * TECROWS, TECROWS), :])

    @pl.when(c == 1)
    def _():
      pltpu.sync_copy(acc.at[pl.ds(s * TECROWS, TECROWS), :],
                      out_hi.at[pl.ds(s * TECROWS, TECROWS), :])

  return run(sd2d, tlo, thi)


def _sc_edge_gather(pqc, i0, i1):
  CH = 112
  NCH = ELTEC // CH

  @functools.partial(
      pl.kernel,
      out_type=jax.ShapeDtypeStruct((ELPAD,), jnp.float32),
      mesh=_mesh(),
      compiler_params=pltpu.CompilerParams(use_tc_tiling_on_sc=False,
                                           needs_layout_passes=False),
      scratch_types=[
          pltpu.VMEM((ELTEC,), jnp.int32),
          pltpu.VMEM((ELTEC,), jnp.int32),
          pltpu.VMEM((ELTEC,), jnp.float32),
          pltpu.VMEM((2 * CH, D2), jnp.float32),
          pltpu.VMEM((2 * CH, D2), jnp.float32),
          pltpu.SemaphoreType.DMA,
      ],
  )
  def run(pq_hbm, i0_hbm, i1_hbm, out_hbm, i0b, i1b, ob, r0, r1, gsem):
    c = lax.axis_index("c")
    s = lax.axis_index("s")
    base = (s * NC + c) * ELTEC
    pltpu.sync_copy(i0_hbm.at[pl.ds(base, ELTEC)], i0b)
    pltpu.sync_copy(i1_hbm.at[pl.ds(base, ELTEC)], i1b)

    def fire(k):
      s2 = lax.rem(k, 2)
      pltpu.async_copy(pq_hbm.at[i0b.at[pl.ds(k * CH, CH)]],
                       r0.at[pl.ds(s2 * CH, CH), :], gsem)
      pltpu.async_copy(pq_hbm.at[i1b.at[pl.ds(k * CH, CH)]],
                       r1.at[pl.ds(s2 * CH, CH), :], gsem)

    def drain(k):
      s2 = lax.rem(k, 2)
      pltpu.make_async_copy(pq_hbm.at[pl.ds(0, CH), :],
                            r0.at[pl.ds(s2 * CH, CH), :], gsem).wait()
      pltpu.make_async_copy(pq_hbm.at[pl.ds(0, CH), :],
                            r1.at[pl.ds(s2 * CH, CH), :], gsem).wait()

    fire(0)
    lane = lax.broadcasted_iota(jnp.int32, (16,), 0)
    zeros = lane * 0
    sixteens = zeros + 16

    def body(k, carry):
      s2 = lax.rem(k, 2)
      drain(k)

      @pl.when(k + 1 < NCH)
      def _():
        fire(k + 1)

      for j in range(CH // 16):
        rows = lane + (s2 * CH + j * 16)
        val = (plsc.load_gather(r0, [rows, zeros])
               + plsc.load_gather(r1, [rows, sixteens]))
        ob[pl.ds(k * CH + j * 16, 16)] = val
      return carry

    lax.fori_loop(0, NCH, body, 0)
    pltpu.sync_copy(ob, out_hbm.at[pl.ds(base, ELTEC)])

  return run(pqc, i0, i1)


def _bd(w32):
  return jnp.kron(jnp.eye(4, dtype=jnp.float32), w32)


def kernel(x_titles_inputs, x_abstracts_inputs, edge_index, edge_label_index,
           Wt, bt, Wa, ba, W1l, W1r, b1, W2l, W2r, b2, Wc, bc):
  src = _perm(edge_index[0])
  dst = _perm(edge_index[1])
  padn = EPAD - E
  pad_src = (jnp.arange(padn, dtype=jnp.int32) * 17) % N
  pad_dst = _perm(N + jnp.arange(padn, dtype=jnp.int32) % (NPAD - N))
  src2d = jnp.concatenate([src, pad_src]).reshape(EROWS, 128)
  dst2d = jnp.concatenate([dst, pad_dst]).reshape(EROWS, 128)
  sd2d = jnp.stack([src2d, dst2d], axis=1)

  xlo, xhi = _tc_embed(x_titles_inputs, x_abstracts_inputs, Wt, bt, Wa, ba)

  cnt0, cnt1 = _sc_counts(sd2d)
  cnt1d = cnt0[:, 0] + cnt1[:, 0]
  cntp = jnp.repeat(cnt1d.reshape(NPROWS, 4), D2, axis=1)

  agg1lo, agg1hi = _sc_scatter(sd2d, xlo.reshape(NPAD, D2),
                               xhi.reshape(NPAD, D2))

  w1 = [_bd(W1l[:D2, :D2]), _bd(W1l[D2:, :D2]),
        _bd(W1r[:D2, :D2]), _bd(W1r[D2:, :D2]),
        _bd(W1l[:D2, D2:]), _bd(W1l[D2:, D2:]),
        _bd(W1r[:D2, D2:]), _bd(W1r[D2:, D2:])]
  b1lo = jnp.tile(b1[:D2], 4).reshape(1, 128)
  b1hi = jnp.tile(b1[D2:], 4).reshape(1, 128)
  h1lo, h1hi = _tc_sage(agg1lo.reshape(NPROWS, 128),
                        agg1hi.reshape(NPROWS, 128),
                        cntp, xlo, xhi, w1, b1lo, b1hi, relu=True)

  agg2lo, agg2hi = _sc_scatter(sd2d, h1lo.reshape(NPAD, D2),
                               h1hi.reshape(NPAD, D2))

  wcu = Wc[:OUT]
  wcv = Wc[OUT:]
  u = W2l @ wcu
  v = W2l @ wcv
  r_ = W2r @ wcu
  t_ = W2r @ wcv

  def arrange(col_u, col_v):
    base = jnp.zeros((D2, D2), jnp.float32)
    base = base.at[:, 0].set(col_u[:, 0]).at[:, 16].set(col_v[:, 0])
    return _bd(base)

  zmat = jnp.zeros((128, 128), jnp.float32)
  w2m = [arrange(u[:D2], v[:D2]), arrange(u[D2:], v[D2:]),
         arrange(r_[:D2], t_[:D2]), arrange(r_[D2:], t_[D2:]),
         zmat, zmat, zmat, zmat]
  c_p = (b2 @ wcu + bc)[0]
  c_q = (b2 @ wcv)[0]
  cvec = jnp.zeros((D2,), jnp.float32).at[0].set(c_p).at[16].set(c_q)
  cvec = jnp.tile(cvec, 4).reshape(1, 128)
  pq, _ = _tc_sage(agg2lo.reshape(NPROWS, 128),
                   agg2hi.reshape(NPROWS, 128),
                   cntp, h1lo, h1hi, w2m, cvec,
                   jnp.zeros((1, 128), jnp.float32), relu=False)

  i0 = jnp.pad(_perm(edge_label_index[0]), (0, ELPAD - EL))
  i1 = jnp.pad(_perm(edge_label_index[1]), (0, ELPAD - EL))
  outf = _sc_edge_gather(pq.reshape(NPAD, D2), i0, i1)
  return outf[:EL].reshape(EL, 1)


# speedup vs baseline: 1.1204x; 1.0002x over previous
import functools

import jax
import jax.numpy as jnp
from jax import lax
from jax.experimental import pallas as pl
from jax.experimental.pallas import tpu as pltpu
from jax.experimental.pallas import tpu_sc as plsc

N = 50000
E = 800000
EL = 100000
D2 = 32
H = 64
OUT = 128

NC = 2
NS = 16
NPAD = 50176
TECROWS = NPAD // NS
NPROWS = NPAD // 4
EROWS = 6272
EPAD = EROWS * 128
RPC = 2
TEC_EROWS = EROWS // NS
CHUNKS = TEC_EROWS // RPC
ELPAD = 100352
ELTEC = ELPAD // (NC * NS)
NP8 = NPAD


def _mesh():
  return plsc.VectorSubcoreMesh(core_axis_name="c", subcore_axis_name="s",
                                num_cores=NC, num_subcores=NS)


def _perm(i):
  b = i // 1024
  w = i % 1024
  return b * 1024 + (w % 256) * 4 + w // 256


def _tc_embed(xt, xa, wt, bt, wa, ba):
  B = 1024

  def body(xt_ref, xa_ref, wt_ref, bt_ref, wa_ref, ba_ref, lo_ref, hi_ref):
    ylo = jnp.dot(xt_ref[...], wt_ref[...],
                  preferred_element_type=jnp.float32) + bt_ref[...]
    yhi = jnp.dot(xa_ref[...], wa_ref[...],
                  preferred_element_type=jnp.float32) + ba_ref[...]
    for a in range(4):
      lo_ref[:, 32 * a:32 * (a + 1)] = ylo[256 * a:256 * (a + 1), :]
      hi_ref[:, 32 * a:32 * (a + 1)] = yhi[256 * a:256 * (a + 1), :]

  return pl.pallas_call(
      body,
      grid=(NPAD // B,),
      in_specs=[
          pl.BlockSpec((B, 768), lambda i: (i, 0)),
          pl.BlockSpec((B, 768), lambda i: (i, 0)),
          pl.BlockSpec((768, D2), lambda i: (0, 0)),
          pl.BlockSpec((1, D2), lambda i: (0, 0)),
          pl.BlockSpec((768, D2), lambda i: (0, 0)),
          pl.BlockSpec((1, D2), lambda i: (0, 0)),
      ],
      out_specs=[pl.BlockSpec((256, 128), lambda i: (i, 0))] * 2,
      out_shape=[jax.ShapeDtypeStruct((NPROWS, 128), jnp.float32)] * 2,
  )(xt, xa, wt, bt.reshape(1, D2), wa, ba.reshape(1, D2))


def _tc_sage(agglo, agghi, cntp, slo, shi, w_mats, b_lo, b_hi, relu):
  B = 256

  def body(al, ah, ct, xl, xh, m0, m1, m2, m3, m4, m5, m6, m7, bl, bh,
           ol, oh):
    inv = 1.0 / jnp.maximum(ct[...], 1.0)
    ml = al[...] * inv
    mh = ah[...] * inv
    dot = lambda x, w: jnp.dot(x, w[...], preferred_element_type=jnp.float32)
    hl = dot(ml, m0) + dot(mh, m1) + dot(xl[...], m2) + dot(xh[...], m3) + bl[...]
    hh = dot(ml, m4) + dot(mh, m5) + dot(xl[...], m6) + dot(xh[...], m7) + bh[...]
    if relu:
      hl = jnp.maximum(hl, 0.0)
      hh = jnp.maximum(hh, 0.0)
    ol[...] = hl
    oh[...] = hh

  row = lambda i: (i, 0)
  fixed = lambda i: (0, 0)
  wspec = pl.BlockSpec((128, 128), fixed)
  return pl.pallas_call(
      body,
      grid=(NPROWS // B,),
      in_specs=[pl.BlockSpec((B, 128), row)] * 5 + [wspec] * 8 +
               [pl.BlockSpec((1, 128), fixed)] * 2,
      out_specs=[pl.BlockSpec((B, 128), row)] * 2,
      out_shape=[jax.ShapeDtypeStruct((NPROWS, 128), jnp.float32)] * 2,
  )(agglo, agghi, cntp, slo, shi, *w_mats, b_lo, b_hi)


def _sc_counts(sd2d):
  @functools.partial(
      pl.kernel,
      out_type=(jax.ShapeDtypeStruct((NPAD, 16), jnp.float32),
                jax.ShapeDtypeStruct((NPAD, 16), jnp.float32)),
      mesh=_mesh(),
      compiler_params=pltpu.CompilerParams(use_tc_tiling_on_sc=False),
      scratch_types=[
          pltpu.VMEM_SHARED((NPAD, 16), jnp.float32),
          pltpu.VMEM((128, 16), jnp.float32),
          pltpu.VMEM((4, 4, 2, 128), jnp.int32),
          pltpu.VMEM((196, 16), jnp.float32),
          pltpu.SemaphoreType.DMA,
          pltpu.SemaphoreType.DMA,
      ],
  )
  def run(dst_hbm, out0, out1, cnt, ones, didx, zbuf, isem, ssem):
    c = lax.axis_index("c")
    s = lax.axis_index("s")
    one = jnp.ones((16,), jnp.float32)
    zero = jnp.zeros((16,), jnp.float32)

    def fill_ones(i, carry):
      ones[i, pl.ds(0, 16)] = one
      return carry

    lax.fori_loop(0, 128, fill_ones, 0)

    def fill_zero(i, carry):
      zbuf[i, pl.ds(0, 16)] = zero
      return carry

    lax.fori_loop(0, 196, fill_zero, 0)

    def zero_acc(t, carry):
      pltpu.sync_copy(zbuf, cnt.at[pl.ds(s * TECROWS + t * 196, 196), :])
      return carry

    lax.fori_loop(0, 16, zero_acc, 0)
    plsc.subcore_barrier()

    base = (c * NS + s) * (EROWS // (NC * NS))

    def fire_idx(k):
      pltpu.async_copy(dst_hbm.at[pl.ds(base + k * 4, 4), :, :],
                       didx.at[lax.rem(k, 4)], isem)

    def wait_idx(k):
      pltpu.make_async_copy(dst_hbm.at[pl.ds(0, 4), :, :],
                            didx.at[lax.rem(k, 4)], isem).wait()

    def fire_scatters(k):
      s4 = lax.rem(k, 4)
      for j in range(4):
        pltpu.async_copy(ones, cnt.at[didx.at[s4, j, 1]], ssem, add=True)

    def drain(k):
      s4 = lax.rem(k, 4)
      for j in range(4):
        pltpu.make_async_copy(ones, cnt.at[didx.at[s4, j, 1]], ssem).wait()

    fire_idx(0)
    fire_idx(1)

    def body(k, carry):
      wait_idx(k)

      @pl.when(k >= 1)
      def _():
        drain(k - 1)

      @pl.when(k + 2 < 49)
      def _():
        fire_idx(k + 2)

      fire_scatters(k)
      return carry

    lax.fori_loop(0, 49, body, 0)
    drain(48)
    plsc.subcore_barrier()

    @pl.when(c == 0)
    def _():
      pltpu.sync_copy(cnt.at[pl.ds(s * TECROWS, TECROWS), :],
                      out0.at[pl.ds(s * TECROWS, TECROWS), :])

    @pl.when(c == 1)
    def _():
      pltpu.sync_copy(cnt.at[pl.ds(s * TECROWS, TECROWS), :],
                      out1.at[pl.ds(s * TECROWS, TECROWS), :])

  return run(sd2d)


def _sc_scatter(sd2d, tlo, thi):
  @functools.partial(
      pl.kernel,
      out_type=(jax.ShapeDtypeStruct((NPAD, D2), jnp.float32),
                jax.ShapeDtypeStruct((NPAD, D2), jnp.float32)),
      mesh=_mesh(),
      compiler_params=pltpu.CompilerParams(use_tc_tiling_on_sc=False),
      scratch_types=[
          pltpu.VMEM_SHARED((NPAD, D2), jnp.float32),
          pltpu.VMEM((4, RPC, 2, 128), jnp.int32),
          pltpu.VMEM((2, RPC * 128, D2), jnp.float32),
          pltpu.SemaphoreType.DMA,
          pltpu.SemaphoreType.DMA,
          pltpu.SemaphoreType.DMA,
      ],
  )
  def run(sd_hbm, tlo_hbm, thi_hbm, out_lo, out_hi,
          acc, idx, rows, isem, gsem, ssem):
    c = lax.axis_index("c")
    s = lax.axis_index("s")
    zero = jnp.zeros((16,), jnp.float32)

    def fill_zero(i, carry):
      rows[0, i, pl.ds(0, 16)] = zero
      rows[0, i, pl.ds(16, 16)] = zero
      rows[1, i, pl.ds(0, 16)] = zero
      rows[1, i, pl.ds(16, 16)] = zero
      return carry

    lax.fori_loop(0, RPC * 128, fill_zero, 0)
    nz = TECROWS // (RPC * 128)

    def zero_acc(t, carry):
      pltpu.sync_copy(rows.at[0],
                      acc.at[pl.ds(s * TECROWS + t * (RPC * 128), RPC * 128), :])
      return carry

    lax.fori_loop(0, nz, zero_acc, 0)
    rem = TECROWS - nz * RPC * 128
    if rem:
      pltpu.sync_copy(rows.at[1, pl.ds(0, rem), :],
                      acc.at[pl.ds(s * TECROWS + nz * RPC * 128, rem), :])
    plsc.subcore_barrier()

    base = s * TEC_EROWS

    def fire_idx(k):
      pltpu.async_copy(sd_hbm.at[pl.ds(base + k * RPC, RPC), :, :],
                       idx.at[lax.rem(k, 4)], isem)

    def wait_idx(k):
      pltpu.make_async_copy(sd_hbm.at[pl.ds(0, RPC), :, :],
                            idx.at[lax.rem(k, 4)], isem).wait()

    def fire_gathers(tbl, k):
      s4 = lax.rem(k, 4)
      s2 = lax.rem(k, 2)
      for j in range(RPC):
        pltpu.async_copy(tbl.at[idx.at[s4, j, 0]],
                         rows.at[s2, pl.ds(j * 128, 128), :], gsem)

    def fire_gathers_c(k):
      @pl.when(c == 0)
      def _():
        fire_gathers(tlo_hbm, k)

      @pl.when(c == 1)
      def _():
        fire_gathers(thi_hbm, k)

    def drain(k, sem):
      s2 = lax.rem(k, 2)
      pltpu.make_async_copy(tlo_hbm.at[pl.ds(0, RPC * 128), :],
                            rows.at[s2], sem).wait()

    def fire_scatters(k):
      s4 = lax.rem(k, 4)
      s2 = lax.rem(k, 2)
      for j in range(RPC):
        pltpu.async_copy(rows.at[s2, pl.ds(j * 128, 128), :],
                         acc.at[idx.at[s4, j, 1]], ssem, add=True)

    fire_idx(0)
    fire_idx(1)
    wait_idx(0)
    fire_gathers_c(0)

    def body(k, carry):
      drain(k, gsem)

      @pl.when(k >= 1)
      def _():
        drain(k - 1, ssem)

      @pl.when(k + 2 < CHUNKS)
      def _():
        fire_idx(k + 2)

      @pl.when(k + 1 < CHUNKS)
      def _():
        wait_idx(k + 1)
        fire_gathers_c(k + 1)

      fire_scatters(k)
      return carry

    lax.fori_loop(0, CHUNKS, body, 0)
    drain(CHUNKS - 1, ssem)
    plsc.subcore_barrier()

    @pl.when(c == 0)
    def _():
      pltpu.sync_copy(acc.at[pl.ds(s * TECROWS, TECROWS), :],
                      out_lo.at[pl.ds(s * TECROWS, TECROWS), :])

    @pl.when(c == 1)
    def _():
      pltpu.sync_copy(acc.at[pl.ds(s * TECROWS, TECROWS), :],
                      out_hi.at[pl.ds(s * TECROWS, TECROWS), :])

  return run(sd2d, tlo, thi)


def _sc_edge_gather(pqc, i0, i1):
  CH = 112
  NCH = ELTEC // CH

  @functools.partial(
      pl.kernel,
      out_type=jax.ShapeDtypeStruct((ELPAD,), jnp.float32),
      mesh=_mesh(),
      compiler_params=pltpu.CompilerParams(use_tc_tiling_on_sc=False,
                                           needs_layout_passes=False),
      scratch_types=[
          pltpu.VMEM((ELTEC,), jnp.int32),
          pltpu.VMEM((ELTEC,), jnp.int32),
          pltpu.VMEM((ELTEC,), jnp.float32),
          pltpu.VMEM((2 * CH, D2), jnp.float32),
          pltpu.VMEM((2 * CH, D2), jnp.float32),
          pltpu.SemaphoreType.DMA,
      ],
  )
  def run(pq_hbm, i0_hbm, i1_hbm, out_hbm, i0b, i1b, ob, r0, r1, gsem):
    c = lax.axis_index("c")
    s = lax.axis_index("s")
    base = (s * NC + c) * ELTEC
    pltpu.sync_copy(i0_hbm.at[pl.ds(base, ELTEC)], i0b)
    pltpu.sync_copy(i1_hbm.at[pl.ds(base, ELTEC)], i1b)

    def fire(k):
      s2 = lax.rem(k, 2)
      pltpu.async_copy(pq_hbm.at[i0b.at[pl.ds(k * CH, CH)]],
                       r0.at[pl.ds(s2 * CH, CH), :], gsem)
      pltpu.async_copy(pq_hbm.at[i1b.at[pl.ds(k * CH, CH)]],
                       r1.at[pl.ds(s2 * CH, CH), :], gsem)

    def drain(k):
      s2 = lax.rem(k, 2)
      pltpu.make_async_copy(pq_hbm.at[pl.ds(0, CH), :],
                            r0.at[pl.ds(s2 * CH, CH), :], gsem).wait()
      pltpu.make_async_copy(pq_hbm.at[pl.ds(0, CH), :],
                            r1.at[pl.ds(s2 * CH, CH), :], gsem).wait()

    fire(0)
    lane = lax.broadcasted_iota(jnp.int32, (16,), 0)
    zeros = lane * 0
    sixteens = zeros + 16

    def body(k, carry):
      s2 = lax.rem(k, 2)
      drain(k)

      @pl.when(k + 1 < NCH)
      def _():
        fire(k + 1)

      for j in range(CH // 16):
        rows = lane + (s2 * CH + j * 16)
        val = (plsc.load_gather(r0, [rows, zeros])
               + plsc.load_gather(r1, [rows, sixteens]))
        ob[pl.ds(k * CH + j * 16, 16)] = val
      return carry

    lax.fori_loop(0, NCH, body, 0)
    pltpu.sync_copy(ob, out_hbm.at[pl.ds(base, ELTEC)])

  return run(pqc, i0, i1)


def _bd(w32):
  return jnp.kron(jnp.eye(4, dtype=jnp.float32), w32)


def kernel(x_titles_inputs, x_abstracts_inputs, edge_index, edge_label_index,
           Wt, bt, Wa, ba, W1l, W1r, b1, W2l, W2r, b2, Wc, bc):
  src = _perm(edge_index[0])
  dst = _perm(edge_index[1])
  padn = EPAD - E
  pad_src = (jnp.arange(padn, dtype=jnp.int32) * 17) % N
  pad_dst = _perm(N + jnp.arange(padn, dtype=jnp.int32) % (NPAD - N))
  src2d = jnp.concatenate([src, pad_src]).reshape(EROWS, 128)
  dst2d = jnp.concatenate([dst, pad_dst]).reshape(EROWS, 128)
  sd2d = jnp.stack([src2d, dst2d], axis=1)

  xlo, xhi = _tc_embed(x_titles_inputs, x_abstracts_inputs, Wt, bt, Wa, ba)

  cnt0, cnt1 = _sc_counts(sd2d)
  cnt1d = cnt0[:, 0] + cnt1[:, 0]
  cntp = jnp.repeat(cnt1d.reshape(NPROWS, 4), D2, axis=1)

  agg1lo, agg1hi = _sc_scatter(sd2d, xlo.reshape(NPAD, D2),
                               xhi.reshape(NPAD, D2))

  w1 = [_bd(W1l[:D2, :D2]), _bd(W1l[D2:, :D2]),
        _bd(W1r[:D2, :D2]), _bd(W1r[D2:, :D2]),
        _bd(W1l[:D2, D2:]), _bd(W1l[D2:, D2:]),
        _bd(W1r[:D2, D2:]), _bd(W1r[D2:, D2:])]
  b1lo = jnp.tile(b1[:D2], 4).reshape(1, 128)
  b1hi = jnp.tile(b1[D2:], 4).reshape(1, 128)
  h1lo, h1hi = _tc_sage(agg1lo.reshape(NPROWS, 128),
                        agg1hi.reshape(NPROWS, 128),
                        cntp, xlo, xhi, w1, b1lo, b1hi, relu=True)

  agg2lo, agg2hi = _sc_scatter(sd2d, h1lo.reshape(NPAD, D2),
                               h1hi.reshape(NPAD, D2))

  wcu = Wc[:OUT]
  wcv = Wc[OUT:]
  u = W2l @ wcu
  v = W2l @ wcv
  r_ = W2r @ wcu
  t_ = W2r @ wcv

  def arrange(col_u, col_v):
    base = jnp.zeros((D2, D2), jnp.float32)
    base = base.at[:, 0].set(col_u[:, 0]).at[:, 16].set(col_v[:, 0])
    return _bd(base)

  zmat = jnp.zeros((128, 128), jnp.float32)
  w2m = [arrange(u[:D2], v[:D2]), arrange(u[D2:], v[D2:]),
         arrange(r_[:D2], t_[:D2]), arrange(r_[D2:], t_[D2:]),
         zmat, zmat, zmat, zmat]
  c_p = (b2 @ wcu + bc)[0]
  c_q = (b2 @ wcv)[0]
  cvec = jnp.zeros((D2,), jnp.float32).at[0].set(c_p).at[16].set(c_q)
  cvec = jnp.tile(cvec, 4).reshape(1, 128)
  pq, _ = _tc_sage(agg2lo.reshape(NPROWS, 128),
                   agg2hi.reshape(NPROWS, 128),
                   cntp, h1lo, h1hi, w2m, cvec,
                   jnp.zeros((1, 128), jnp.float32), relu=False)

  i0 = jnp.pad(_perm(edge_label_index[0]), (0, ELPAD - EL))
  i1 = jnp.pad(_perm(edge_label_index[1]), (0, ELPAD - EL))
  outf = _sc_edge_gather(pq.reshape(NPAD, D2), i0, i1)
  return outf[:EL].reshape(EL, 1)
